# baseline (speedup 1.0000x reference)
import math

import jax
import jax.numpy as jnp
from jax.experimental import pallas as pl
from jax.experimental.pallas import tpu as pltpu

_B, _N, _DIN, _D = 4, 4096, 128, 128
_H = 4
_DH = _D // _H
_GS = (8, 8)
_IMG = (256.0, 256.0)
_NB = 1
_NC = _GS[0] * _GS[1]
_K = (2 * _NB + 1) ** 2

_BLK = 2048
_NBLK = _N // _BLK
_SCALE = 1.0 / math.sqrt(_DH)


def _cell_xy(co):
    cw = _IMG[0] / _GS[0]
    ch = _IMG[1] / _GS[1]
    gx = jnp.clip((co[:, 0:1] / cw).astype(jnp.int32), 0, _GS[0] - 1)
    gy = jnp.clip((co[:, 1:2] / ch).astype(jnp.int32), 0, _GS[1] - 1)
    return gx, gy


def _finish(feat, o, any_valid, lng, lnb, out_ref):
    enh = feat + jnp.where(any_valid, o, 0.0)
    mu = jnp.mean(enh, axis=1, keepdims=True)
    var = jnp.mean((enh - mu) ** 2, axis=1, keepdims=True)
    out_ref[0] = (enh - mu) / jnp.sqrt(var + 1e-5) * lng + lnb


def _fused_kernel(x_ref, c_ref, offs_ref, wf_ref, bf_ref,
                  wp1_ref, bp1_ref, wp2_ref, bp2_ref,
                  wq_ref, bq_ref, wk_ref, bk_ref, wv_ref, bv_ref,
                  wo_ref, bo_ref, lng_ref, lnb_ref, out_ref,
                  kcell_ref, vcell_ref, pk_ref, pv_ref, occ_ref,
                  r1_ref, row1_ref, r2_ref, wob_ref, val_ref, meta_ref):
    f32 = jnp.float32
    i32 = jnp.int32
    n = pl.program_id(1)
    dn_t = (((1,), (1,)), ((), ()))
    dn_n = (((1,), (0,)), ((), ()))

    @pl.when(n == 0)
    def _tables():
        x = x_ref[0]
        co = c_ref[0]
        gx, gy = _cell_xy(co)
        cell = gx * _GS[1] + gy
        lane = jax.lax.broadcasted_iota(i32, (_N, _NC), 1)
        oh = (cell == lane).astype(f32)
        dn0 = (((0,), (0,)), ((), ()))
        csum = jax.lax.dot_general(oh, x, dn0, preferred_element_type=f32)
        cntr = jnp.sum(oh, axis=0, keepdims=True)
        cntc = jnp.transpose(cntr)
        occ_ref[...] = cntr

        csum_feat = (jnp.dot(csum, wf_ref[...], preferred_element_type=f32)
                     + cntc * bf_ref[...])
        cmean = csum_feat / jnp.maximum(cntc, 1.0)
        kcell = (jnp.dot(cmean, wk_ref[...], preferred_element_type=f32)
                 + bk_ref[...])
        vcell = (jnp.dot(cmean, wv_ref[...], preferred_element_type=f32)
                 + bv_ref[...])
        pe = jnp.maximum(
            jnp.dot(offs_ref[...], wp1_ref[...], preferred_element_type=f32)
            + bp1_ref[...], 0.0)
        pe = jnp.dot(pe, wp2_ref[...], preferred_element_type=f32) + bp2_ref[...]
        pk = jnp.dot(pe, wk_ref[...], preferred_element_type=f32)
        pv = jnp.dot(pe, wv_ref[...], preferred_element_type=f32)
        kcell_ref[...] = kcell
        vcell_ref[...] = vcell
        pk_ref[...] = pk
        pv_ref[...] = pv

        pgx = gx[0:1, 0:1]
        pgy = gy[0:1, 0:1]
        ri = jax.lax.broadcasted_iota(i32, (16, _NC), 0)
        ci = jax.lax.broadcasted_iota(i32, (16, _NC), 1)
        dxj = ri // (2 * _NB + 1) - _NB
        dyj = ri % (2 * _NB + 1) - _NB
        nx = pgx + dxj
        ny = pgy + dyj
        inb = ((nx >= 0) & (nx < _GS[0]) & (ny >= 0) & (ny < _GS[1])
               & (ri < _K))
        sel = (inb & (ci == nx * _GS[1] + ny)).astype(f32)
        k9 = jnp.dot(sel, kcell, preferred_element_type=f32) + pk
        v9 = jnp.dot(sel, vcell, preferred_element_type=f32) + pv
        l16 = jax.lax.broadcasted_iota(i32, (16, _D), 1)
        k4 = jnp.concatenate(
            [jnp.where(l16 // _DH == h, k9, 0.0) for h in range(_H)], axis=0)
        v4 = jnp.concatenate(
            [jnp.where(l16 // _DH == h, v9, 0.0) for h in range(_H)], axis=0)
        a4 = jax.lax.dot_general(
            wq_ref[...], k4, dn_t, preferred_element_type=f32) * _SCALE
        c4 = jax.lax.dot_general(
            bq_ref[...], k4, dn_t, preferred_element_type=f32) * _SCALE
        a4x = jnp.dot(wf_ref[...], a4, preferred_element_type=f32)
        c4x = c4 + jax.lax.dot_general(bf_ref[...], a4, dn_n,
                                       preferred_element_type=f32)
        bf16 = jnp.bfloat16
        r1_ref[...] = jnp.concatenate([wf_ref[...], a4x],
                                      axis=1).astype(bf16)
        row1_ref[...] = jnp.concatenate([bf_ref[...], c4x], axis=1)
        ri64 = jax.lax.broadcasted_iota(i32, (4 * 16, 4 * 16), 0)
        cj64 = jax.lax.broadcasted_iota(i32, (4 * 16, 4 * 16), 1)
        d2 = ((ri64 // 16) == cj64).astype(f32)
        r2_ref[...] = jnp.concatenate([v4, d2], axis=1).astype(bf16)
        wob_ref[...] = wo_ref[...].astype(bf16)

        occf = (cntr > 0.0).astype(f32)
        sel4 = jnp.concatenate([sel, sel, sel, sel], axis=0)
        occrow = jax.lax.dot_general(occf, sel4, dn_t,
                                     preferred_element_type=f32)
        validf = (occrow > 0.0).astype(f32)
        val_ref[...] = validf
        anyv = jnp.max(validf, axis=1, keepdims=True)
        unif = (jnp.max(cntr, axis=1, keepdims=True) == f32(_N)).astype(f32)
        l128 = jax.lax.broadcasted_iota(i32, (1, _D), 1)
        meta_ref[...] = (jnp.where(l128 == 0, unif, 0.0)
                         + jnp.where(l128 == 1, anyv, 0.0))

    @pl.when(n > 0)
    def _attend():
        start = (n - 1) * _BLK
        x = x_ref[0, pl.ds(start, _BLK), :]
        lng = lng_ref[...]
        lnb = lnb_ref[...]
        neg = f32(-1e9)

        meta = meta_ref[...]
        uniform = jnp.min(meta[0:1, 0:1]) > 0.0

        @pl.when(uniform)
        def _fast():
            bf16 = jnp.bfloat16
            y1 = (jax.lax.dot_general(x.astype(bf16), r1_ref[...], dn_n,
                                      preferred_element_type=f32)
                  + row1_ref[...])
            feat = y1[:, 0:_D]
            s = y1[:, _D:_D + 4 * 16]
            validrow = val_ref[...] > 0.0
            s = jnp.where(validrow, s, neg)
            m = jnp.max(s, axis=1, keepdims=True)
            e = jnp.exp(s - m)
            y2 = jax.lax.dot_general(e.astype(bf16), r2_ref[...], dn_n,
                                     preferred_element_type=f32)
            out_nf = y2[:, 0:_D]
            lane = jax.lax.broadcasted_iota(i32, (1, _D), 1)
            denom = y2[:, _D + 3:_D + 4]
            for h in range(_H - 2, -1, -1):
                denom = jnp.where(lane < (h + 1) * _DH,
                                  y2[:, _D + h:_D + h + 1], denom)
            out = out_nf / denom
            o = (jnp.dot(out.astype(bf16), wob_ref[...],
                         preferred_element_type=f32) + bo_ref[...])
            any_valid = meta[0:1, 1:2] > 0.0
            _finish(feat, o, any_valid, lng, lnb, out_ref)

        @pl.when(jnp.logical_not(uniform))
        def _general():
            feat = (jnp.dot(x, wf_ref[...], preferred_element_type=f32)
                    + bf_ref[...])
            kcell = kcell_ref[...]
            vcell = vcell_ref[...]
            pk = pk_ref[...]
            pv = pv_ref[...]
            occ = occ_ref[...] > 0.0
            co = c_ref[0, pl.ds(start, _BLK), :]
            gx, gy = _cell_xy(co)
            q = (jnp.dot(feat, wq_ref[...], preferred_element_type=f32)
                 + bq_ref[...])
            lane_c = jax.lax.broadcasted_iota(i32, (_BLK, _NC), 1)
            cx = lane_c // _GS[1]
            cy = lane_c % _GS[1]
            dx = cx - gx
            dy = cy - gy
            geo = (jnp.abs(dx) <= _NB) & (jnp.abs(dy) <= _NB)
            valid = geo & occ
            jmap = (dx + _NB) * (2 * _NB + 1) + (dy + _NB)
            scale = f32(_SCALE)

            lane_d = jax.lax.broadcasted_iota(i32, (_NC, _D), 1)
            lane_d16 = jax.lax.broadcasted_iota(i32, (16, _D), 1)
            out = jnp.zeros((_BLK, _D), f32)
            for h in range(_H):
                mask_c = (lane_d // _DH == h).astype(f32)
                mask_p = (lane_d16 // _DH == h).astype(f32)
                s = jax.lax.dot_general(q, kcell * mask_c, dn_t,
                                        preferred_element_type=f32)
                qp = jax.lax.dot_general(q, pk * mask_p, dn_t,
                                         preferred_element_type=f32)
                pos_s = jnp.zeros((_BLK, _NC), f32)
                for j in range(_K):
                    pos_s = pos_s + jnp.where(jmap == j, qp[:, j:j + 1], 0.0)
                s = (s + pos_s) * scale
                s = jnp.where(valid, s, neg)
                m = jnp.max(s, axis=1, keepdims=True)
                e = jnp.exp(s - m)
                attn = e / jnp.sum(e, axis=1, keepdims=True)
                out = out + jax.lax.dot_general(attn, vcell * mask_c, dn_n,
                                                preferred_element_type=f32)
                pvh = pv * mask_p
                for j in range(_K):
                    aj = jnp.sum(jnp.where(jmap == j, attn, 0.0), axis=1,
                                 keepdims=True)
                    out = out + aj * pvh[j:j + 1, :]

            o = (jnp.dot(out, wo_ref[...], preferred_element_type=f32)
                 + bo_ref[...])
            any_valid = jnp.max(valid.astype(f32), axis=1, keepdims=True) > 0.0
            _finish(feat, o, any_valid, lng, lnb, out_ref)


def kernel(features, coords, W_feat, b_feat, Wp1, bp1, Wp2, bp2, Wq, bq,
           Wk, bk, Wv, bv, Wo, bo, ln_g, ln_b):
    f32 = jnp.float32
    row = lambda v: v.reshape(1, -1).astype(f32)
    full = lambda shape: pl.BlockSpec(shape, lambda b, n: tuple(0 for _ in shape))

    offs = jnp.zeros((16, 2), f32)
    offs_list = [[float(dx), float(dy)]
                 for dx in range(-_NB, _NB + 1) for dy in range(-_NB, _NB + 1)]
    offs = offs.at[:_K].set(jnp.array(offs_list, f32))

    out = pl.pallas_call(
        _fused_kernel,
        grid=(_B, 1 + _NBLK),
        in_specs=[
            pl.BlockSpec((1, _N, _DIN), lambda b, n: (b, 0, 0)),
            pl.BlockSpec((1, _N, 2), lambda b, n: (b, 0, 0)),
            full((16, 2)),
            full((_DIN, _D)), full((1, _D)),
            full((2, _D // 2)), full((1, _D // 2)),
            full((_D // 2, _D)), full((1, _D)),
            full((_D, _D)), full((1, _D)),
            full((_D, _D)), full((1, _D)),
            full((_D, _D)), full((1, _D)),
            full((_D, _D)), full((1, _D)),
            full((1, _D)), full((1, _D)),
        ],
        out_specs=pl.BlockSpec(
            (1, _BLK, _D), lambda b, n: (b, jnp.maximum(n - 1, 0), 0)),
        out_shape=jax.ShapeDtypeStruct((_B, _N, _D), f32),
        scratch_shapes=[
            pltpu.VMEM((_NC, _D), f32), pltpu.VMEM((_NC, _D), f32),
            pltpu.VMEM((16, _D), f32), pltpu.VMEM((16, _D), f32),
            pltpu.VMEM((1, _NC), f32),
            pltpu.VMEM((_DIN, _D + 4 * 16), jnp.bfloat16),
            pltpu.VMEM((1, _D + 4 * 16), f32),
            pltpu.VMEM((4 * 16, _D + 4 * 16), jnp.bfloat16),
            pltpu.VMEM((_D, _D), jnp.bfloat16),
            pltpu.VMEM((1, 4 * 16), f32),
            pltpu.VMEM((1, _D), f32),
        ],
        compiler_params=pltpu.CompilerParams(
            dimension_semantics=("parallel", "arbitrary")),
    )(features, coords, offs, W_feat, row(b_feat), Wp1, row(bp1), Wp2,
      row(bp2), Wq, row(bq), Wk, row(bk), Wv, row(bv), Wo, row(bo),
      row(ln_g), row(ln_b))
    return out


# speedup vs baseline: 1.0012x; 1.0012x over previous
import math

import jax
import jax.numpy as jnp
from jax.experimental import pallas as pl
from jax.experimental.pallas import tpu as pltpu

_B, _N, _DIN, _D = 4, 4096, 128, 128
_H = 4
_DH = _D // _H
_GS = (8, 8)
_IMG = (256.0, 256.0)
_NB = 1
_NC = _GS[0] * _GS[1]
_K = (2 * _NB + 1) ** 2

_BLK = 2048
_NBLK = _N // _BLK
_SCALE = 1.0 / math.sqrt(_DH)


def _cell_xy(co):
    cw = _IMG[0] / _GS[0]
    ch = _IMG[1] / _GS[1]
    gx = jnp.clip((co[:, 0:1] / cw).astype(jnp.int32), 0, _GS[0] - 1)
    gy = jnp.clip((co[:, 1:2] / ch).astype(jnp.int32), 0, _GS[1] - 1)
    return gx, gy


def _finish(feat, o, any_valid, lng, lnb, out_ref):
    enh = feat + jnp.where(any_valid, o, 0.0)
    mu = jnp.mean(enh, axis=1, keepdims=True)
    var = jnp.mean((enh - mu) ** 2, axis=1, keepdims=True)
    out_ref[0] = (enh - mu) / jnp.sqrt(var + 1e-5) * lng + lnb


def _fused_kernel(x_ref, c_ref, offs_ref, wf_ref, bf_ref,
                  wp1_ref, bp1_ref, wp2_ref, bp2_ref,
                  wq_ref, bq_ref, wk_ref, bk_ref, wv_ref, bv_ref,
                  wo_ref, bo_ref, lng_ref, lnb_ref, out_ref,
                  kcell_ref, vcell_ref, pk_ref, pv_ref, occ_ref,
                  r1_ref, row1_ref, r2_ref, val_ref, meta_ref):
    f32 = jnp.float32
    i32 = jnp.int32
    n = pl.program_id(1)
    dn_t = (((1,), (1,)), ((), ()))
    dn_n = (((1,), (0,)), ((), ()))

    @pl.when(n == 0)
    def _tables():
        x = x_ref[0]
        co = c_ref[0]
        gx, gy = _cell_xy(co)
        cell = gx * _GS[1] + gy
        lane = jax.lax.broadcasted_iota(i32, (_N, _NC), 1)
        oh = (cell == lane).astype(f32)
        dn0 = (((0,), (0,)), ((), ()))
        csum = jax.lax.dot_general(oh, x, dn0, preferred_element_type=f32)
        cntr = jnp.sum(oh, axis=0, keepdims=True)
        cntc = jnp.transpose(cntr)
        occ_ref[...] = cntr

        csum_feat = (jnp.dot(csum, wf_ref[...], preferred_element_type=f32)
                     + cntc * bf_ref[...])
        cmean = csum_feat / jnp.maximum(cntc, 1.0)
        kcell = (jnp.dot(cmean, wk_ref[...], preferred_element_type=f32)
                 + bk_ref[...])
        vcell = (jnp.dot(cmean, wv_ref[...], preferred_element_type=f32)
                 + bv_ref[...])
        pe = jnp.maximum(
            jnp.dot(offs_ref[...], wp1_ref[...], preferred_element_type=f32)
            + bp1_ref[...], 0.0)
        pe = jnp.dot(pe, wp2_ref[...], preferred_element_type=f32) + bp2_ref[...]
        pk = jnp.dot(pe, wk_ref[...], preferred_element_type=f32)
        pv = jnp.dot(pe, wv_ref[...], preferred_element_type=f32)
        kcell_ref[...] = kcell
        vcell_ref[...] = vcell
        pk_ref[...] = pk
        pv_ref[...] = pv

        pgx = gx[0:1, 0:1]
        pgy = gy[0:1, 0:1]
        ri = jax.lax.broadcasted_iota(i32, (16, _NC), 0)
        ci = jax.lax.broadcasted_iota(i32, (16, _NC), 1)
        dxj = ri // (2 * _NB + 1) - _NB
        dyj = ri % (2 * _NB + 1) - _NB
        nx = pgx + dxj
        ny = pgy + dyj
        inb = ((nx >= 0) & (nx < _GS[0]) & (ny >= 0) & (ny < _GS[1])
               & (ri < _K))
        sel = (inb & (ci == nx * _GS[1] + ny)).astype(f32)
        k9 = jnp.dot(sel, kcell, preferred_element_type=f32) + pk
        v9 = jnp.dot(sel, vcell, preferred_element_type=f32) + pv
        l16 = jax.lax.broadcasted_iota(i32, (16, _D), 1)
        k4 = jnp.concatenate(
            [jnp.where(l16 // _DH == h, k9, 0.0) for h in range(_H)], axis=0)
        v4 = jnp.concatenate(
            [jnp.where(l16 // _DH == h, v9, 0.0) for h in range(_H)], axis=0)
        a4 = jax.lax.dot_general(
            wq_ref[...], k4, dn_t, preferred_element_type=f32) * _SCALE
        c4 = jax.lax.dot_general(
            bq_ref[...], k4, dn_t, preferred_element_type=f32) * _SCALE
        a4x = jnp.dot(wf_ref[...], a4, preferred_element_type=f32)
        c4x = c4 + jax.lax.dot_general(bf_ref[...], a4, dn_n,
                                       preferred_element_type=f32)
        r1_ref[...] = jnp.concatenate([wf_ref[...], a4x], axis=1)
        row1_ref[...] = jnp.concatenate([bf_ref[...], c4x], axis=1)
        ri64 = jax.lax.broadcasted_iota(i32, (4 * 16, 4 * 16), 0)
        cj64 = jax.lax.broadcasted_iota(i32, (4 * 16, 4 * 16), 1)
        d2 = ((ri64 // 16) == cj64).astype(f32)
        r2_ref[...] = jnp.concatenate([v4, d2], axis=1)

        occf = (cntr > 0.0).astype(f32)
        sel4 = jnp.concatenate([sel, sel, sel, sel], axis=0)
        occrow = jax.lax.dot_general(occf, sel4, dn_t,
                                     preferred_element_type=f32)
        validf = (occrow > 0.0).astype(f32)
        val_ref[...] = validf
        anyv = jnp.max(validf, axis=1, keepdims=True)
        unif = (jnp.max(cntr, axis=1, keepdims=True) == f32(_N)).astype(f32)
        l128 = jax.lax.broadcasted_iota(i32, (1, _D), 1)
        meta_ref[...] = (jnp.where(l128 == 0, unif, 0.0)
                         + jnp.where(l128 == 1, anyv, 0.0))

    @pl.when(n > 0)
    def _attend():
        start = (n - 1) * _BLK
        x = x_ref[0, pl.ds(start, _BLK), :]
        lng = lng_ref[...]
        lnb = lnb_ref[...]
        neg = f32(-1e9)

        meta = meta_ref[...]
        uniform = jnp.min(meta[0:1, 0:1]) > 0.0

        @pl.when(uniform)
        def _fast():
            bf16 = jnp.bfloat16
            y1 = (jax.lax.dot_general(x.astype(bf16),
                                      r1_ref[...].astype(bf16), dn_n,
                                      preferred_element_type=f32)
                  + row1_ref[...])
            feat = y1[:, 0:_D]
            s = y1[:, _D:_D + 4 * 16]
            validrow = val_ref[...] > 0.0
            s = jnp.where(validrow, s, neg)
            m = jnp.max(s, axis=1, keepdims=True)
            e = jnp.exp(s - m)
            y2 = jax.lax.dot_general(e.astype(bf16),
                                     r2_ref[...].astype(bf16), dn_n,
                                     preferred_element_type=f32)
            out_nf = y2[:, 0:_D]
            lane = jax.lax.broadcasted_iota(i32, (1, _D), 1)
            denom = y2[:, _D + 3:_D + 4]
            for h in range(_H - 2, -1, -1):
                denom = jnp.where(lane < (h + 1) * _DH,
                                  y2[:, _D + h:_D + h + 1], denom)
            out = out_nf / denom
            o = (jnp.dot(out.astype(bf16), wo_ref[...].astype(bf16),
                         preferred_element_type=f32) + bo_ref[...])
            any_valid = meta[0:1, 1:2] > 0.0
            _finish(feat, o, any_valid, lng, lnb, out_ref)

        @pl.when(jnp.logical_not(uniform))
        def _general():
            feat = (jnp.dot(x, wf_ref[...], preferred_element_type=f32)
                    + bf_ref[...])
            kcell = kcell_ref[...]
            vcell = vcell_ref[...]
            pk = pk_ref[...]
            pv = pv_ref[...]
            occ = occ_ref[...] > 0.0
            co = c_ref[0, pl.ds(start, _BLK), :]
            gx, gy = _cell_xy(co)
            q = (jnp.dot(feat, wq_ref[...], preferred_element_type=f32)
                 + bq_ref[...])
            lane_c = jax.lax.broadcasted_iota(i32, (_BLK, _NC), 1)
            cx = lane_c // _GS[1]
            cy = lane_c % _GS[1]
            dx = cx - gx
            dy = cy - gy
            geo = (jnp.abs(dx) <= _NB) & (jnp.abs(dy) <= _NB)
            valid = geo & occ
            jmap = (dx + _NB) * (2 * _NB + 1) + (dy + _NB)
            scale = f32(_SCALE)

            lane_d = jax.lax.broadcasted_iota(i32, (_NC, _D), 1)
            lane_d16 = jax.lax.broadcasted_iota(i32, (16, _D), 1)
            out = jnp.zeros((_BLK, _D), f32)
            for h in range(_H):
                mask_c = (lane_d // _DH == h).astype(f32)
                mask_p = (lane_d16 // _DH == h).astype(f32)
                s = jax.lax.dot_general(q, kcell * mask_c, dn_t,
                                        preferred_element_type=f32)
                qp = jax.lax.dot_general(q, pk * mask_p, dn_t,
                                         preferred_element_type=f32)
                pos_s = jnp.zeros((_BLK, _NC), f32)
                for j in range(_K):
                    pos_s = pos_s + jnp.where(jmap == j, qp[:, j:j + 1], 0.0)
                s = (s + pos_s) * scale
                s = jnp.where(valid, s, neg)
                m = jnp.max(s, axis=1, keepdims=True)
                e = jnp.exp(s - m)
                attn = e / jnp.sum(e, axis=1, keepdims=True)
                out = out + jax.lax.dot_general(attn, vcell * mask_c, dn_n,
                                                preferred_element_type=f32)
                pvh = pv * mask_p
                for j in range(_K):
                    aj = jnp.sum(jnp.where(jmap == j, attn, 0.0), axis=1,
                                 keepdims=True)
                    out = out + aj * pvh[j:j + 1, :]

            o = (jnp.dot(out, wo_ref[...], preferred_element_type=f32)
                 + bo_ref[...])
            any_valid = jnp.max(valid.astype(f32), axis=1, keepdims=True) > 0.0
            _finish(feat, o, any_valid, lng, lnb, out_ref)


def kernel(features, coords, W_feat, b_feat, Wp1, bp1, Wp2, bp2, Wq, bq,
           Wk, bk, Wv, bv, Wo, bo, ln_g, ln_b):
    f32 = jnp.float32
    row = lambda v: v.reshape(1, -1).astype(f32)
    full = lambda shape: pl.BlockSpec(shape, lambda b, n: tuple(0 for _ in shape))

    offs = jnp.zeros((16, 2), f32)
    offs_list = [[float(dx), float(dy)]
                 for dx in range(-_NB, _NB + 1) for dy in range(-_NB, _NB + 1)]
    offs = offs.at[:_K].set(jnp.array(offs_list, f32))

    out = pl.pallas_call(
        _fused_kernel,
        grid=(_B, 1 + _NBLK),
        in_specs=[
            pl.BlockSpec((1, _N, _DIN), lambda b, n: (b, 0, 0)),
            pl.BlockSpec((1, _N, 2), lambda b, n: (b, 0, 0)),
            full((16, 2)),
            full((_DIN, _D)), full((1, _D)),
            full((2, _D // 2)), full((1, _D // 2)),
            full((_D // 2, _D)), full((1, _D)),
            full((_D, _D)), full((1, _D)),
            full((_D, _D)), full((1, _D)),
            full((_D, _D)), full((1, _D)),
            full((_D, _D)), full((1, _D)),
            full((1, _D)), full((1, _D)),
        ],
        out_specs=pl.BlockSpec(
            (1, _BLK, _D), lambda b, n: (b, jnp.maximum(n - 1, 0), 0)),
        out_shape=jax.ShapeDtypeStruct((_B, _N, _D), f32),
        scratch_shapes=[
            pltpu.VMEM((_NC, _D), f32), pltpu.VMEM((_NC, _D), f32),
            pltpu.VMEM((16, _D), f32), pltpu.VMEM((16, _D), f32),
            pltpu.VMEM((1, _NC), f32),
            pltpu.VMEM((_DIN, _D + 4 * 16), f32),
            pltpu.VMEM((1, _D + 4 * 16), f32),
            pltpu.VMEM((4 * 16, _D + 4 * 16), f32),
            pltpu.VMEM((1, 4 * 16), f32),
            pltpu.VMEM((1, _D), f32),
        ],
        compiler_params=pltpu.CompilerParams(
            dimension_semantics=("parallel", "arbitrary")),
    )(features, coords, offs, W_feat, row(b_feat), Wp1, row(bp1), Wp2,
      row(bp2), Wq, row(bq), Wk, row(bk), Wv, row(bv), Wo, row(bo),
      row(ln_g), row(ln_b))
    return out


# speedup vs baseline: 1.0031x; 1.0019x over previous
import math

import jax
import jax.numpy as jnp
from jax.experimental import pallas as pl
from jax.experimental.pallas import tpu as pltpu

_B, _N, _DIN, _D = 4, 4096, 128, 128
_H = 4
_DH = _D // _H
_GS = (8, 8)
_IMG = (256.0, 256.0)
_NB = 1
_NC = _GS[0] * _GS[1]
_K = (2 * _NB + 1) ** 2

_BLK = 2048
_NBLK = _N // _BLK
_SCALE = 1.0 / math.sqrt(_DH)


def _cell_xy(co):
    cw = _IMG[0] / _GS[0]
    ch = _IMG[1] / _GS[1]
    gx = jnp.clip((co[:, 0:1] / cw).astype(jnp.int32), 0, _GS[0] - 1)
    gy = jnp.clip((co[:, 1:2] / ch).astype(jnp.int32), 0, _GS[1] - 1)
    return gx, gy


def _finish(feat, o, any_valid, lng, lnb, out_ref):
    enh = feat + jnp.where(any_valid, o, 0.0)
    mu = jnp.mean(enh, axis=1, keepdims=True)
    var = jnp.mean((enh - mu) ** 2, axis=1, keepdims=True)
    out_ref[0] = (enh - mu) / jnp.sqrt(var + 1e-5) * lng + lnb


def _fused_kernel(x_ref, c_ref, offs_ref, wf_ref, bf_ref,
                  wp1_ref, bp1_ref, wp2_ref, bp2_ref,
                  wq_ref, bq_ref, wk_ref, bk_ref, wv_ref, bv_ref,
                  wo_ref, bo_ref, lng_ref, lnb_ref, out_ref,
                  kcell_ref, vcell_ref, pk_ref, pv_ref, occ_ref,
                  r1_ref, row1_ref, r2_ref, val_ref, meta_ref):
    f32 = jnp.float32
    i32 = jnp.int32
    n = pl.program_id(1)
    dn_t = (((1,), (1,)), ((), ()))
    dn_n = (((1,), (0,)), ((), ()))

    @pl.when(n == 0)
    def _tables():
        x = x_ref[0]
        co = c_ref[0]
        gx, gy = _cell_xy(co)
        cell = gx * _GS[1] + gy
        lane = jax.lax.broadcasted_iota(i32, (_N, _NC), 1)
        oh = (cell == lane).astype(f32)
        dn0 = (((0,), (0,)), ((), ()))
        csum = jax.lax.dot_general(oh, x, dn0, preferred_element_type=f32)
        cntr = jnp.sum(oh, axis=0, keepdims=True)
        cntc = jnp.transpose(cntr)
        occ_ref[...] = cntr

        csum_feat = (jnp.dot(csum, wf_ref[...], preferred_element_type=f32)
                     + cntc * bf_ref[...])
        cmean = csum_feat / jnp.maximum(cntc, 1.0)
        kcell = (jnp.dot(cmean, wk_ref[...], preferred_element_type=f32)
                 + bk_ref[...])
        vcell = (jnp.dot(cmean, wv_ref[...], preferred_element_type=f32)
                 + bv_ref[...])
        pe = jnp.maximum(
            jnp.dot(offs_ref[...], wp1_ref[...], preferred_element_type=f32)
            + bp1_ref[...], 0.0)
        pe = jnp.dot(pe, wp2_ref[...], preferred_element_type=f32) + bp2_ref[...]
        pk = jnp.dot(pe, wk_ref[...], preferred_element_type=f32)
        pv = jnp.dot(pe, wv_ref[...], preferred_element_type=f32)
        kcell_ref[...] = kcell
        vcell_ref[...] = vcell
        pk_ref[...] = pk
        pv_ref[...] = pv

        pgx = gx[0:1, 0:1]
        pgy = gy[0:1, 0:1]
        ri = jax.lax.broadcasted_iota(i32, (16, _NC), 0)
        ci = jax.lax.broadcasted_iota(i32, (16, _NC), 1)
        dxj = ri // (2 * _NB + 1) - _NB
        dyj = ri % (2 * _NB + 1) - _NB
        nx = pgx + dxj
        ny = pgy + dyj
        inb = ((nx >= 0) & (nx < _GS[0]) & (ny >= 0) & (ny < _GS[1])
               & (ri < _K))
        sel = (inb & (ci == nx * _GS[1] + ny)).astype(f32)
        k9 = jnp.dot(sel, kcell, preferred_element_type=f32) + pk
        v9 = jnp.dot(sel, vcell, preferred_element_type=f32) + pv
        l16 = jax.lax.broadcasted_iota(i32, (16, _D), 1)
        k4 = jnp.concatenate(
            [jnp.where(l16 // _DH == h, k9, 0.0) for h in range(_H)], axis=0)
        v4 = jnp.concatenate(
            [jnp.where(l16 // _DH == h, v9, 0.0) for h in range(_H)], axis=0)
        a4 = jax.lax.dot_general(
            wq_ref[...], k4, dn_t, preferred_element_type=f32) * _SCALE
        c4 = jax.lax.dot_general(
            bq_ref[...], k4, dn_t, preferred_element_type=f32) * _SCALE
        a4x = jnp.dot(wf_ref[...], a4, preferred_element_type=f32)
        c4x = c4 + jax.lax.dot_general(bf_ref[...], a4, dn_n,
                                       preferred_element_type=f32)
        r1_ref[...] = jnp.concatenate([wf_ref[...], a4x], axis=1)
        row1_ref[...] = jnp.concatenate([bf_ref[...], c4x], axis=1)
        ri64 = jax.lax.broadcasted_iota(i32, (4 * 16, 4 * 16), 0)
        cj64 = jax.lax.broadcasted_iota(i32, (4 * 16, 4 * 16), 1)
        d2 = ((ri64 // 16) == cj64).astype(f32)
        r2_ref[...] = jnp.concatenate([v4, d2], axis=1)

        occf = (cntr > 0.0).astype(f32)
        sel4 = jnp.concatenate([sel, sel, sel, sel], axis=0)
        occrow = jax.lax.dot_general(occf, sel4, dn_t,
                                     preferred_element_type=f32)
        validf = (occrow > 0.0).astype(f32)
        val_ref[...] = validf
        anyv = jnp.max(validf, axis=1, keepdims=True)
        unif = (jnp.max(cntr, axis=1, keepdims=True) == f32(_N)).astype(f32)
        l128 = jax.lax.broadcasted_iota(i32, (1, _D), 1)
        meta_ref[...] = (jnp.where(l128 == 0, unif, 0.0)
                         + jnp.where(l128 == 1, anyv, 0.0))

    @pl.when(n > 0)
    def _attend():
        start = (n - 1) * _BLK
        x = x_ref[0, pl.ds(start, _BLK), :]
        lng = lng_ref[...]
        lnb = lnb_ref[...]
        neg = f32(-1e9)

        meta = meta_ref[...]
        uniform = jnp.min(meta[0:1, 0:1]) > 0.0

        @pl.when(uniform)
        def _fast():
            y1 = (jax.lax.dot_general(x, r1_ref[...], dn_n,
                                      preferred_element_type=f32)
                  + row1_ref[...])
            feat = y1[:, 0:_D]
            s = y1[:, _D:_D + 4 * 16]
            validrow = val_ref[...] > 0.0
            s = jnp.where(validrow, s, neg)
            m = jnp.max(s, axis=1, keepdims=True)
            e = jnp.exp(s - m)
            y2 = jax.lax.dot_general(e, r2_ref[...], dn_n,
                                     preferred_element_type=f32)
            out_nf = y2[:, 0:_D]
            lane = jax.lax.broadcasted_iota(i32, (1, _D), 1)
            denom = y2[:, _D + 3:_D + 4]
            for h in range(_H - 2, -1, -1):
                denom = jnp.where(lane < (h + 1) * _DH,
                                  y2[:, _D + h:_D + h + 1], denom)
            out = out_nf / denom
            o = (jnp.dot(out, wo_ref[...],
                         preferred_element_type=f32) + bo_ref[...])
            any_valid = meta[0:1, 1:2] > 0.0
            _finish(feat, o, any_valid, lng, lnb, out_ref)

        @pl.when(jnp.logical_not(uniform))
        def _general():
            feat = (jnp.dot(x, wf_ref[...], preferred_element_type=f32)
                    + bf_ref[...])
            kcell = kcell_ref[...]
            vcell = vcell_ref[...]
            pk = pk_ref[...]
            pv = pv_ref[...]
            occ = occ_ref[...] > 0.0
            co = c_ref[0, pl.ds(start, _BLK), :]
            gx, gy = _cell_xy(co)
            q = (jnp.dot(feat, wq_ref[...], preferred_element_type=f32)
                 + bq_ref[...])
            lane_c = jax.lax.broadcasted_iota(i32, (_BLK, _NC), 1)
            cx = lane_c // _GS[1]
            cy = lane_c % _GS[1]
            dx = cx - gx
            dy = cy - gy
            geo = (jnp.abs(dx) <= _NB) & (jnp.abs(dy) <= _NB)
            valid = geo & occ
            jmap = (dx + _NB) * (2 * _NB + 1) + (dy + _NB)
            scale = f32(_SCALE)

            lane_d = jax.lax.broadcasted_iota(i32, (_NC, _D), 1)
            lane_d16 = jax.lax.broadcasted_iota(i32, (16, _D), 1)
            out = jnp.zeros((_BLK, _D), f32)
            for h in range(_H):
                mask_c = (lane_d // _DH == h).astype(f32)
                mask_p = (lane_d16 // _DH == h).astype(f32)
                s = jax.lax.dot_general(q, kcell * mask_c, dn_t,
                                        preferred_element_type=f32)
                qp = jax.lax.dot_general(q, pk * mask_p, dn_t,
                                         preferred_element_type=f32)
                pos_s = jnp.zeros((_BLK, _NC), f32)
                for j in range(_K):
                    pos_s = pos_s + jnp.where(jmap == j, qp[:, j:j + 1], 0.0)
                s = (s + pos_s) * scale
                s = jnp.where(valid, s, neg)
                m = jnp.max(s, axis=1, keepdims=True)
                e = jnp.exp(s - m)
                attn = e / jnp.sum(e, axis=1, keepdims=True)
                out = out + jax.lax.dot_general(attn, vcell * mask_c, dn_n,
                                                preferred_element_type=f32)
                pvh = pv * mask_p
                for j in range(_K):
                    aj = jnp.sum(jnp.where(jmap == j, attn, 0.0), axis=1,
                                 keepdims=True)
                    out = out + aj * pvh[j:j + 1, :]

            o = (jnp.dot(out, wo_ref[...], preferred_element_type=f32)
                 + bo_ref[...])
            any_valid = jnp.max(valid.astype(f32), axis=1, keepdims=True) > 0.0
            _finish(feat, o, any_valid, lng, lnb, out_ref)


def kernel(features, coords, W_feat, b_feat, Wp1, bp1, Wp2, bp2, Wq, bq,
           Wk, bk, Wv, bv, Wo, bo, ln_g, ln_b):
    f32 = jnp.float32
    row = lambda v: v.reshape(1, -1).astype(f32)
    full = lambda shape: pl.BlockSpec(shape, lambda b, n: tuple(0 for _ in shape))

    offs = jnp.zeros((16, 2), f32)
    offs_list = [[float(dx), float(dy)]
                 for dx in range(-_NB, _NB + 1) for dy in range(-_NB, _NB + 1)]
    offs = offs.at[:_K].set(jnp.array(offs_list, f32))

    out = pl.pallas_call(
        _fused_kernel,
        grid=(_B, 1 + _NBLK),
        in_specs=[
            pl.BlockSpec((1, _N, _DIN), lambda b, n: (b, 0, 0)),
            pl.BlockSpec((1, _N, 2), lambda b, n: (b, 0, 0)),
            full((16, 2)),
            full((_DIN, _D)), full((1, _D)),
            full((2, _D // 2)), full((1, _D // 2)),
            full((_D // 2, _D)), full((1, _D)),
            full((_D, _D)), full((1, _D)),
            full((_D, _D)), full((1, _D)),
            full((_D, _D)), full((1, _D)),
            full((_D, _D)), full((1, _D)),
            full((1, _D)), full((1, _D)),
        ],
        out_specs=pl.BlockSpec(
            (1, _BLK, _D), lambda b, n: (b, jnp.maximum(n - 1, 0), 0)),
        out_shape=jax.ShapeDtypeStruct((_B, _N, _D), f32),
        scratch_shapes=[
            pltpu.VMEM((_NC, _D), f32), pltpu.VMEM((_NC, _D), f32),
            pltpu.VMEM((16, _D), f32), pltpu.VMEM((16, _D), f32),
            pltpu.VMEM((1, _NC), f32),
            pltpu.VMEM((_DIN, _D + 4 * 16), f32),
            pltpu.VMEM((1, _D + 4 * 16), f32),
            pltpu.VMEM((4 * 16, _D + 4 * 16), f32),
            pltpu.VMEM((1, 4 * 16), f32),
            pltpu.VMEM((1, _D), f32),
        ],
        compiler_params=pltpu.CompilerParams(
            dimension_semantics=("parallel", "arbitrary")),
    )(features, coords, offs, W_feat, row(b_feat), Wp1, row(bp1), Wp2,
      row(bp2), Wq, row(bq), Wk, row(bk), Wv, row(bv), Wo, row(bo),
      row(ln_g), row(ln_b))
    return out


# speedup vs baseline: 1.5469x; 1.5420x over previous
import math

import jax
import jax.numpy as jnp
from jax.experimental import pallas as pl
from jax.experimental.pallas import tpu as pltpu

_B, _N, _DIN, _D = 4, 4096, 128, 128
_H = 4
_DH = _D // _H
_GS = (8, 8)
_IMG = (256.0, 256.0)
_NB = 1
_NC = _GS[0] * _GS[1]
_K = (2 * _NB + 1) ** 2

_BLK = 2048
_NBLK = _N // _BLK
_SCALE = 1.0 / math.sqrt(_DH)


def _cell_xy(co):
    cw = _IMG[0] / _GS[0]
    ch = _IMG[1] / _GS[1]
    gx = jnp.clip((co[:, 0:1] / cw).astype(jnp.int32), 0, _GS[0] - 1)
    gy = jnp.clip((co[:, 1:2] / ch).astype(jnp.int32), 0, _GS[1] - 1)
    return gx, gy


def _finish(feat, o, any_valid, lng, lnb, out_ref):
    enh = feat + jnp.where(any_valid, o, 0.0)
    mu = jnp.mean(enh, axis=1, keepdims=True)
    var = jnp.mean((enh - mu) ** 2, axis=1, keepdims=True)
    out_ref[0] = (enh - mu) / jnp.sqrt(var + 1e-5) * lng + lnb


def _fused_kernel(x_ref, c_ref, offs_ref, wf_ref, bf_ref,
                  wp1_ref, bp1_ref, wp2_ref, bp2_ref,
                  wq_ref, bq_ref, wk_ref, bk_ref, wv_ref, bv_ref,
                  wo_ref, bo_ref, lng_ref, lnb_ref, out_ref,
                  kcell_ref, vcell_ref, pk_ref, pv_ref, occ_ref,
                  a4_ref, c4_ref, vw4_ref, val_ref, meta_ref):
    f32 = jnp.float32
    i32 = jnp.int32
    n = pl.program_id(1)
    dn_t = (((1,), (1,)), ((), ()))
    dn_n = (((1,), (0,)), ((), ()))

    @pl.when(n == 0)
    def _tables():
        x = x_ref[0]
        co = c_ref[0]
        gx, gy = _cell_xy(co)
        cell = gx * _GS[1] + gy
        lane = jax.lax.broadcasted_iota(i32, (_N, _NC), 1)
        oh = (cell == lane).astype(f32)
        dn0 = (((0,), (0,)), ((), ()))
        csum = jax.lax.dot_general(oh, x, dn0, preferred_element_type=f32)
        cntr = jnp.sum(oh, axis=0, keepdims=True)
        cntc = jnp.transpose(cntr)
        occ_ref[...] = cntr

        csum_feat = (jnp.dot(csum, wf_ref[...], preferred_element_type=f32)
                     + cntc * bf_ref[...])
        cmean = csum_feat / jnp.maximum(cntc, 1.0)
        kcell = (jnp.dot(cmean, wk_ref[...], preferred_element_type=f32)
                 + bk_ref[...])
        vcell = (jnp.dot(cmean, wv_ref[...], preferred_element_type=f32)
                 + bv_ref[...])
        pe = jnp.maximum(
            jnp.dot(offs_ref[...], wp1_ref[...], preferred_element_type=f32)
            + bp1_ref[...], 0.0)
        pe = jnp.dot(pe, wp2_ref[...], preferred_element_type=f32) + bp2_ref[...]
        pk = jnp.dot(pe, wk_ref[...], preferred_element_type=f32)
        pv = jnp.dot(pe, wv_ref[...], preferred_element_type=f32)
        kcell_ref[...] = kcell
        vcell_ref[...] = vcell
        pk_ref[...] = pk
        pv_ref[...] = pv

        pgx = gx[0:1, 0:1]
        pgy = gy[0:1, 0:1]
        ri = jax.lax.broadcasted_iota(i32, (16, _NC), 0)
        ci = jax.lax.broadcasted_iota(i32, (16, _NC), 1)
        dxj = ri // (2 * _NB + 1) - _NB
        dyj = ri % (2 * _NB + 1) - _NB
        nx = pgx + dxj
        ny = pgy + dyj
        inb = ((nx >= 0) & (nx < _GS[0]) & (ny >= 0) & (ny < _GS[1])
               & (ri < _K))
        sel = (inb & (ci == nx * _GS[1] + ny)).astype(f32)
        k9 = jnp.dot(sel, kcell, preferred_element_type=f32) + pk
        v9 = jnp.dot(sel, vcell, preferred_element_type=f32) + pv
        l16 = jax.lax.broadcasted_iota(i32, (16, _D), 1)
        k4 = jnp.concatenate(
            [jnp.where(l16 // _DH == h, k9, 0.0) for h in range(_H)], axis=0)
        v4 = jnp.concatenate(
            [jnp.where(l16 // _DH == h, v9, 0.0) for h in range(_H)], axis=0)
        a4_ref[...] = jax.lax.dot_general(
            wq_ref[...], k4, dn_t, preferred_element_type=f32) * _SCALE
        c4_ref[...] = jax.lax.dot_general(
            bq_ref[...], k4, dn_t, preferred_element_type=f32) * _SCALE
        vw4_ref[...] = jnp.dot(v4, wo_ref[...], preferred_element_type=f32)

        occf = (cntr > 0.0).astype(f32)
        sel4 = jnp.concatenate([sel, sel, sel, sel], axis=0)
        occrow = jax.lax.dot_general(occf, sel4, dn_t,
                                     preferred_element_type=f32)
        validf = (occrow > 0.0).astype(f32)
        val_ref[...] = validf
        anyv = jnp.max(validf, axis=1, keepdims=True)
        unif = (jnp.max(cntr, axis=1, keepdims=True) == f32(_N)).astype(f32)
        l128 = jax.lax.broadcasted_iota(i32, (1, _D), 1)
        meta_ref[...] = (jnp.where(l128 == 0, unif, 0.0)
                         + jnp.where(l128 == 1, anyv, 0.0))

    @pl.when(n > 0)
    def _attend():
        start = (n - 1) * _BLK
        x = x_ref[0, pl.ds(start, _BLK), :]
        lng = lng_ref[...]
        lnb = lnb_ref[...]
        neg = f32(-1e9)

        meta = meta_ref[...]
        uniform = jnp.min(meta[0:1, 0:1]) > 0.0

        @pl.when(uniform)
        def _fast():
            feat = (jnp.dot(x, wf_ref[...], preferred_element_type=f32)
                    + bf_ref[...])
            s = (jax.lax.dot_general(feat, a4_ref[...], dn_n,
                                     preferred_element_type=f32)
                 + c4_ref[...])
            validrow = val_ref[...] > 0.0
            s = jnp.where(validrow, s, neg)
            m = jnp.max(s, axis=1, keepdims=True)
            e = jnp.exp(s - m)
            gi = jax.lax.broadcasted_iota(i32, (4 * 16, 4 * 16), 0)
            gj = jax.lax.broadcasted_iota(i32, (4 * 16, 4 * 16), 1)
            g16 = ((gi // 16) == (gj // 16)).astype(f32)
            attn = e / jax.lax.dot_general(e, g16, dn_n,
                                           preferred_element_type=f32)
            o = (jax.lax.dot_general(attn, vw4_ref[...], dn_n,
                                     preferred_element_type=f32) + bo_ref[...])
            any_valid = meta[0:1, 1:2] > 0.0
            _finish(feat, o, any_valid, lng, lnb, out_ref)

        @pl.when(jnp.logical_not(uniform))
        def _general():
            feat = (jnp.dot(x, wf_ref[...], preferred_element_type=f32)
                    + bf_ref[...])
            kcell = kcell_ref[...]
            vcell = vcell_ref[...]
            pk = pk_ref[...]
            pv = pv_ref[...]
            occ = occ_ref[...] > 0.0
            co = c_ref[0, pl.ds(start, _BLK), :]
            gx, gy = _cell_xy(co)
            q = (jnp.dot(feat, wq_ref[...], preferred_element_type=f32)
                 + bq_ref[...])
            lane_c = jax.lax.broadcasted_iota(i32, (_BLK, _NC), 1)
            cx = lane_c // _GS[1]
            cy = lane_c % _GS[1]
            dx = cx - gx
            dy = cy - gy
            geo = (jnp.abs(dx) <= _NB) & (jnp.abs(dy) <= _NB)
            valid = geo & occ
            jmap = (dx + _NB) * (2 * _NB + 1) + (dy + _NB)
            scale = f32(_SCALE)

            lane_d = jax.lax.broadcasted_iota(i32, (_NC, _D), 1)
            lane_d16 = jax.lax.broadcasted_iota(i32, (16, _D), 1)
            out = jnp.zeros((_BLK, _D), f32)
            for h in range(_H):
                mask_c = (lane_d // _DH == h).astype(f32)
                mask_p = (lane_d16 // _DH == h).astype(f32)
                s = jax.lax.dot_general(q, kcell * mask_c, dn_t,
                                        preferred_element_type=f32)
                qp = jax.lax.dot_general(q, pk * mask_p, dn_t,
                                         preferred_element_type=f32)
                pos_s = jnp.zeros((_BLK, _NC), f32)
                for j in range(_K):
                    pos_s = pos_s + jnp.where(jmap == j, qp[:, j:j + 1], 0.0)
                s = (s + pos_s) * scale
                s = jnp.where(valid, s, neg)
                m = jnp.max(s, axis=1, keepdims=True)
                e = jnp.exp(s - m)
                attn = e / jnp.sum(e, axis=1, keepdims=True)
                out = out + jax.lax.dot_general(attn, vcell * mask_c, dn_n,
                                                preferred_element_type=f32)
                pvh = pv * mask_p
                for j in range(_K):
                    aj = jnp.sum(jnp.where(jmap == j, attn, 0.0), axis=1,
                                 keepdims=True)
                    out = out + aj * pvh[j:j + 1, :]

            o = (jnp.dot(out, wo_ref[...], preferred_element_type=f32)
                 + bo_ref[...])
            any_valid = jnp.max(valid.astype(f32), axis=1, keepdims=True) > 0.0
            _finish(feat, o, any_valid, lng, lnb, out_ref)


def kernel(features, coords, W_feat, b_feat, Wp1, bp1, Wp2, bp2, Wq, bq,
           Wk, bk, Wv, bv, Wo, bo, ln_g, ln_b):
    f32 = jnp.float32
    row = lambda v: v.reshape(1, -1).astype(f32)
    full = lambda shape: pl.BlockSpec(shape, lambda b, n: tuple(0 for _ in shape))

    offs = jnp.zeros((16, 2), f32)
    offs_list = [[float(dx), float(dy)]
                 for dx in range(-_NB, _NB + 1) for dy in range(-_NB, _NB + 1)]
    offs = offs.at[:_K].set(jnp.array(offs_list, f32))

    out = pl.pallas_call(
        _fused_kernel,
        grid=(_B, 1 + _NBLK),
        in_specs=[
            pl.BlockSpec((1, _N, _DIN), lambda b, n: (b, 0, 0)),
            pl.BlockSpec((1, _N, 2), lambda b, n: (b, 0, 0)),
            full((16, 2)),
            full((_DIN, _D)), full((1, _D)),
            full((2, _D // 2)), full((1, _D // 2)),
            full((_D // 2, _D)), full((1, _D)),
            full((_D, _D)), full((1, _D)),
            full((_D, _D)), full((1, _D)),
            full((_D, _D)), full((1, _D)),
            full((_D, _D)), full((1, _D)),
            full((1, _D)), full((1, _D)),
        ],
        out_specs=pl.BlockSpec(
            (1, _BLK, _D), lambda b, n: (b, jnp.maximum(n - 1, 0), 0)),
        out_shape=jax.ShapeDtypeStruct((_B, _N, _D), f32),
        scratch_shapes=[
            pltpu.VMEM((_NC, _D), f32), pltpu.VMEM((_NC, _D), f32),
            pltpu.VMEM((16, _D), f32), pltpu.VMEM((16, _D), f32),
            pltpu.VMEM((1, _NC), f32),
            pltpu.VMEM((_D, 4 * 16), f32),
            pltpu.VMEM((1, 4 * 16), f32),
            pltpu.VMEM((4 * 16, _D), f32),
            pltpu.VMEM((1, 4 * 16), f32),
            pltpu.VMEM((1, _D), f32),
        ],
        compiler_params=pltpu.CompilerParams(
            dimension_semantics=("parallel", "arbitrary")),
    )(features, coords, offs, W_feat, row(b_feat), Wp1, row(bp1), Wp2,
      row(bp2), Wq, row(bq), Wk, row(bk), Wv, row(bv), Wo, row(bo),
      row(ln_g), row(ln_b))
    return out
